# Initial kernel scaffold; baseline (speedup 1.0000x reference)
import jax
import jax.numpy as jnp
from jax.experimental import pallas as pl


def kernel(x, tables, W1, W2, W3):
    raise NotImplementedError("write your pallas kernel here")


# speedup vs baseline: 31.0132x; 31.0132x over previous
import functools
import math

import jax
import jax.numpy as jnp
from jax import lax
from jax.experimental import pallas as pl
from jax.experimental.pallas import tpu as pltpu
from jax.experimental.pallas import tpu_sc as plsc

_LEVELS = 8
_BASE = 8
_FINEST = 160
_LOG2_T = 19
_T = 1 << _LOG2_T
_F = 2
_SCALE = math.exp(math.log(_FINEST / _BASE) / (_LEVELS - 1))
_RES = [int(math.floor(_BASE * (_SCALE ** l))) for l in range(_LEVELS)]
_P2 = 2654435761 - (1 << 32)
_P3 = 805459861
_MASK = _T - 1

_NC = 2
_NS = 16
_NW = _NC * _NS
_CHUNK = 128


def _encode(xT, tab_flat):
    n = xT.shape[1]
    ppw = n // _NW
    nchunk = ppw // _CHUNK

    mesh = plsc.VectorSubcoreMesh(
        core_axis_name="c", subcore_axis_name="s",
        num_cores=_NC, num_subcores=_NS)

    @functools.partial(
        pl.kernel,
        out_type=jax.ShapeDtypeStruct((_LEVELS * _F, n), jnp.float32),
        mesh=mesh,
        scratch_types=[
            pltpu.VMEM((4, _CHUNK), jnp.float32),
            pltpu.VMEM((8, _CHUNK), jnp.int32),
            pltpu.VMEM((8, _CHUNK), jnp.int32),
            pltpu.VMEM((8, _CHUNK), jnp.float32),
            pltpu.VMEM((8, _CHUNK), jnp.float32),
            pltpu.VMEM((8, _CHUNK), jnp.float32),
            pltpu.VMEM((_LEVELS * _F, _CHUNK), jnp.float32),
            pltpu.SemaphoreType.DMA,
        ],
    )
    def enc_kernel(xT_hbm, tab_hbm, enc_hbm,
                   xs_v, idx0_v, idx1_v, vals0_v, vals1_v, w_v, enc_v, sem):
        wid = lax.axis_index("s") * _NC + lax.axis_index("c")
        base0 = wid * ppw

        def chunk_body(ci, carry):
            base = base0 + ci * _CHUNK
            pltpu.sync_copy(xT_hbm.at[:, pl.ds(base, _CHUNK)],
                            xs_v.at[0:3, :])
            for l in range(_LEVELS):
                res = jnp.float32(_RES[l])
                lbase = l * 2 * _T

                def grp_idx(g, c, l=l, res=res, lbase=lbase):
                    s = pl.ds(g * 16, 16)
                    xv = xs_v[0, s]
                    yv = xs_v[1, s]
                    zv = xs_v[2, s]
                    posx = xv * res
                    posy = yv * res
                    posz = zv * res
                    pxi = posx.astype(jnp.int32)
                    pyi = posy.astype(jnp.int32)
                    pzi = posz.astype(jnp.int32)
                    fx = posx - pxi.astype(jnp.float32)
                    fy = posy - pyi.astype(jnp.float32)
                    fz = posz - pzi.astype(jnp.float32)
                    a = (pxi, pxi + 1)
                    b0 = pyi * jnp.int32(_P2)
                    b = (b0, b0 + jnp.int32(_P2))
                    c0 = pzi * jnp.int32(_P3)
                    cc = (c0, c0 + jnp.int32(_P3))
                    wx = (jnp.float32(1.0) - fx, fx)
                    wy = (jnp.float32(1.0) - fy, fy)
                    wz = (jnp.float32(1.0) - fz, fz)
                    wyz = (wy[0] * wz[0], wy[0] * wz[1],
                           wy[1] * wz[0], wy[1] * wz[1])
                    for corner in range(8):
                        i, j, k = corner >> 2, (corner >> 1) & 1, corner & 1
                        h = (a[i] ^ b[j] ^ cc[k]) & jnp.int32(_MASK)
                        f0 = h + h + jnp.int32(lbase)
                        idx0_v[corner, s] = f0
                        idx1_v[corner, s] = f0 + 1
                        w_v[corner, s] = wx[i] * wyz[2 * j + k]
                    return c

                lax.fori_loop(0, _CHUNK // 16, grp_idx, 0)
                cps = []
                for corner in range(8):
                    cps.append(pltpu.async_copy(
                        tab_hbm.at[idx0_v.at[corner]], vals0_v.at[corner], sem))
                    cps.append(pltpu.async_copy(
                        tab_hbm.at[idx1_v.at[corner]], vals1_v.at[corner], sem))
                for cp in cps:
                    cp.wait()

                def grp_acc(g, c, l=l):
                    s = pl.ds(g * 16, 16)
                    w0 = w_v[0, s]
                    acc0 = w0 * vals0_v[0, s]
                    acc1 = w0 * vals1_v[0, s]
                    for corner in range(1, 8):
                        wc = w_v[corner, s]
                        acc0 = acc0 + wc * vals0_v[corner, s]
                        acc1 = acc1 + wc * vals1_v[corner, s]
                    enc_v[2 * l, s] = acc0
                    enc_v[2 * l + 1, s] = acc1
                    return c

                lax.fori_loop(0, _CHUNK // 16, grp_acc, 0)
            pltpu.sync_copy(enc_v, enc_hbm.at[:, pl.ds(base, _CHUNK)])
            return carry

        lax.fori_loop(0, nchunk, chunk_body, 0)

    return enc_kernel(xT, tab_flat)


_BMLP = 4096


def _mlp_body(enc_ref, w1_ref, w2_ref, w3_ref, out_ref):
    e = enc_ref[...]
    h = jnp.maximum(
        jnp.dot(w1_ref[...], e, preferred_element_type=jnp.float32), 0.0)
    h = jnp.maximum(
        jnp.dot(w2_ref[...], h, preferred_element_type=jnp.float32), 0.0)
    out_ref[...] = jnp.dot(
        w3_ref[...], h, preferred_element_type=jnp.float32)[0]


def _mlp(enc, W1T, W2T, W3T):
    n = enc.shape[1]
    grid = (n // _BMLP,)
    return pl.pallas_call(
        _mlp_body,
        grid=grid,
        in_specs=[
            pl.BlockSpec((_LEVELS * _F, _BMLP), lambda i: (0, i)),
            pl.BlockSpec((64, _LEVELS * _F), lambda i: (0, 0)),
            pl.BlockSpec((64, 64), lambda i: (0, 0)),
            pl.BlockSpec((1, 64), lambda i: (0, 0)),
        ],
        out_specs=pl.BlockSpec((_BMLP,), lambda i: (i,)),
        out_shape=jax.ShapeDtypeStruct((n,), jnp.float32),
    )(enc, W1T, W2T, W3T)


def kernel(x, tables, W1, W2, W3):
    n = x.shape[0]
    xT = x.T
    tab_flat = tables.reshape(-1)
    enc = _encode(xT, tab_flat)
    out = _mlp(enc, W1.T, W2.T, W3.T)
    return out.reshape(x.shape[:-1])
